# baseline (speedup 1.0000x reference)
import functools

import jax
import jax.numpy as jnp
from jax import lax
from jax.experimental import pallas as pl
from jax.experimental.pallas import tpu as pltpu
from jax.experimental.pallas import tpu_sc as plsc

NC = 2
NS = 16
NW = NC * NS
LANES = 16
CB = 4
OUTP = 32


def _lane_shuffle(v, idx):
  dnums = lax.GatherDimensionNumbers(
      offset_dims=(), collapsed_slice_dims=(0,), start_index_map=(0,))
  return lax.gather(v, idx[:, None], dnums, slice_sizes=(1,),
                    mode=lax.GatherScatterMode.PROMISE_IN_BOUNDS)


def _cbow_body(num_chunks, L, K, D,
               ctx_hbm, tgt_hbm, emb_hbm, tgtw_hbm, out_hbm,
               idxc_a, idxt_a, rows_ca, rows_ta, pooled_a, scores_a,
               sem_ca, sem_ta, sem_ia, sem_oa,
               idxc_b, idxt_b, rows_cb, rows_tb, pooled_b, scores_b,
               sem_cb, sem_tb, sem_ib, sem_ob,
               e0_buf, tscr):
  nd = D // LANES
  wid = lax.axis_index("s") * NC + lax.axis_index("c")
  base = wid * (num_chunks * CB)
  iota = lax.iota(jnp.int32, LANES)

  pltpu.sync_copy(emb_hbm.at[pl.ds(0, 1)], e0_buf)
  for r in range(OUTP):
    tscr[r] = jnp.zeros((LANES,), jnp.float32)

  def stage(chunk, idxc, idxt, half, sem_i):
    row0 = base + chunk * CB
    pltpu.async_copy(ctx_hbm.at[pl.ds(row0, CB)],
                     idxc.at[pl.ds(half * CB, CB)], sem_i)
    pltpu.async_copy(tgt_hbm.at[pl.ds(row0, CB)],
                     idxt.at[pl.ds(half * CB, CB)], sem_i)

  def drain_idx(idxc, idxt, half, sem_i):
    pltpu.make_async_copy(ctx_hbm.at[pl.ds(base, CB)],
                          idxc.at[pl.ds(half * CB, CB)], sem_i).wait()
    pltpu.make_async_copy(tgt_hbm.at[pl.ds(base, CB)],
                          idxt.at[pl.ds(half * CB, CB)], sem_i).wait()

  def fire(idxc, idxt, half, rows_c, rows_t, sem_c, sem_t):
    for j in range(CB):
      pltpu.async_copy(emb_hbm.at[idxc.at[half * CB + j]],
                       rows_c.at[pl.ds(j * L, L)], sem_c)
      pltpu.async_copy(tgtw_hbm.at[idxt.at[half * CB + j]],
                       rows_t.at[pl.ds(j * K, K)], sem_t)

  def drain(idxc, idxt, half, rows_c, rows_t, sem_c, sem_t):
    for j in range(CB):
      pltpu.make_async_copy(emb_hbm.at[idxc.at[half * CB + j]],
                            rows_c.at[pl.ds(j * L, L)], sem_c).wait()
      pltpu.make_async_copy(tgtw_hbm.at[idxt.at[half * CB + j]],
                            rows_t.at[pl.ds(j * K, K)], sem_t).wait()

  def compute(chunk, idxc, half, rows_c, rows_t, pooled, scores, sem_o):
    def pool_b(b, carry2):
      rowvec = jnp.full((LANES,), half * CB + b, jnp.int32)
      bvec = jnp.full((LANES,), b, jnp.int32)
      zc = jnp.zeros((LANES,), jnp.float32)
      one = jnp.float32(1.0)
      zero = jnp.float32(0.0)
      for g in range(L // LANES):
        v = plsc.load_gather(idxc, [rowvec, iota + g * LANES])
        zc = zc + jnp.where(v == 0, one, zero)
      rem = L % LANES
      if rem:
        v = plsc.load_gather(idxc, [rowvec, iota + (L - LANES)])
        zc = zc + jnp.where((v == 0) & (iota >= LANES - rem), one, zero)
      nz = zc
      for sh in (8, 4, 2, 1):
        nz = nz + _lane_shuffle(nz, iota ^ sh)

      UNROLL = 5
      def l_body(l, acc):
        rbase = jnp.full((LANES,), b * L + UNROLL * l, jnp.int32)
        for u in range(UNROLL):
          acc = tuple(
              acc[cc] + plsc.load_gather(rows_c, [rbase + u, iota + cc * LANES])
              for cc in range(nd))
        return acc
      acc = lax.fori_loop(0, L // UNROLL, l_body,
                          tuple(jnp.zeros((LANES,), jnp.float32)
                                for _ in range(nd)))
      scale = jnp.float32(1.0 / L)
      for cc in range(nd):
        e0 = e0_buf[0, pl.ds(cc * LANES, LANES)]
        plsc.store_scatter(pooled, [bvec, iota + cc * LANES],
                           (acc[cc] - nz * e0) * scale)
      return carry2

    lax.fori_loop(0, CB, pool_b, 0)

    @pl.when(chunk >= 2)
    def _():
      pltpu.make_async_copy(scores, out_hbm.at[pl.ds(base, CB)], sem_o).wait()

    def score_b(b, carry2):
      bvec = jnp.full((LANES,), b, jnp.int32)
      p = [plsc.load_gather(pooled, [bvec, iota + cc * LANES])
           for cc in range(nd)]
      for k in range(K):
        rvec = jnp.full((LANES,), b * K + k, jnp.int32)
        acc = plsc.load_gather(rows_t, [rvec, iota]) * p[0]
        for cc in range(1, nd):
          acc = acc + plsc.load_gather(
              rows_t, [rvec, iota + cc * LANES]) * p[cc]
        tscr[k] = acc
      for g in range(OUTP // LANES):
        ridx = iota + (g * LANES)
        s = plsc.load_gather(tscr, [ridx, jnp.zeros((LANES,), jnp.int32)])
        for cc in range(1, LANES):
          s = s + plsc.load_gather(
              tscr, [ridx, jnp.full((LANES,), cc, jnp.int32)])
        plsc.store_scatter(scores, [bvec, iota + g * LANES], s)
      return carry2

    lax.fori_loop(0, CB, score_b, 0)
    pltpu.async_copy(scores, out_hbm.at[pl.ds(base + chunk * CB, CB)], sem_o)

  zero = jnp.int32(0)
  stage(zero, idxc_a, idxt_a, zero, sem_ia)
  drain_idx(idxc_a, idxt_a, zero, sem_ia)
  fire(idxc_a, idxt_a, zero, rows_ca, rows_ta, sem_ca, sem_ta)
  stage(jnp.int32(1), idxc_b, idxt_b, zero, sem_ib)

  last = jnp.int32(num_chunks - 1)

  def pair(i, carry):
    ca = 2 * i
    cb = 2 * i + 1
    h = i & 1
    hn = 1 - h
    drain_idx(idxc_b, idxt_b, h, sem_ib)
    fire(idxc_b, idxt_b, h, rows_cb, rows_tb, sem_cb, sem_tb)
    stage(jnp.minimum(ca + 2, last), idxc_a, idxt_a, hn, sem_ia)
    drain(idxc_a, idxt_a, h, rows_ca, rows_ta, sem_ca, sem_ta)
    compute(ca, idxc_a, h, rows_ca, rows_ta, pooled_a, scores_a, sem_oa)
    drain_idx(idxc_a, idxt_a, hn, sem_ia)
    fire(idxc_a, idxt_a, hn, rows_ca, rows_ta, sem_ca, sem_ta)
    stage(jnp.minimum(cb + 2, last), idxc_b, idxt_b, hn, sem_ib)
    drain(idxc_b, idxt_b, h, rows_cb, rows_tb, sem_cb, sem_tb)
    compute(cb, idxc_b, h, rows_cb, rows_tb, pooled_b, scores_b, sem_ob)
    return carry

  lax.fori_loop(0, num_chunks // 2, pair, 0)
  hlast = jnp.int32((num_chunks // 2) & 1)
  drain(idxc_a, idxt_a, hlast, rows_ca, rows_ta, sem_ca, sem_ta)
  drain_idx(idxc_b, idxt_b, hlast, sem_ib)
  pltpu.make_async_copy(scores_a, out_hbm.at[pl.ds(base, CB)], sem_oa).wait()
  pltpu.make_async_copy(scores_b, out_hbm.at[pl.ds(base, CB)], sem_ob).wait()


def kernel(context, target, emb_weight, tgt_weight):
  B, L = context.shape
  _, K = target.shape
  V, D = emb_weight.shape
  assert B % (NW * CB * 2) == 0
  num_chunks = B // (NW * CB)

  body = functools.partial(_cbow_body, num_chunks, L, K, D)
  mesh = plsc.VectorSubcoreMesh(core_axis_name="c", subcore_axis_name="s")

  def set_scratch():
    return [
        pltpu.VMEM((2 * CB, L), jnp.int32),
        pltpu.VMEM((2 * CB, K), jnp.int32),
        pltpu.VMEM((CB * L, D), jnp.float32),
        pltpu.VMEM((CB * K, D), jnp.float32),
        pltpu.VMEM((CB, D), jnp.float32),
        pltpu.VMEM((CB, OUTP), jnp.float32),
        pltpu.SemaphoreType.DMA,
        pltpu.SemaphoreType.DMA,
        pltpu.SemaphoreType.DMA,
        pltpu.SemaphoreType.DMA,
    ]

  run = pl.kernel(
      body,
      out_type=jax.ShapeDtypeStruct((B, OUTP), jnp.float32),
      mesh=mesh,
      compiler_params=pltpu.CompilerParams(needs_layout_passes=False),
      scratch_types=set_scratch() + set_scratch() + [
          pltpu.VMEM((1, D), jnp.float32),
          pltpu.VMEM((OUTP, LANES), jnp.float32),
      ],
  )
  out = run(context.astype(jnp.int32), target.astype(jnp.int32),
            emb_weight, tgt_weight)
  return out[:, :K]


# speedup vs baseline: 1.1525x; 1.1525x over previous
import functools

import jax
import jax.numpy as jnp
from jax import lax
from jax.experimental import pallas as pl
from jax.experimental.pallas import tpu as pltpu
from jax.experimental.pallas import tpu_sc as plsc

NC = 2
NS = 16
NW = NC * NS
LANES = 16
CB = 4
OUTP = 32


def _lane_shuffle(v, idx):
  dnums = lax.GatherDimensionNumbers(
      offset_dims=(), collapsed_slice_dims=(0,), start_index_map=(0,))
  return lax.gather(v, idx[:, None], dnums, slice_sizes=(1,),
                    mode=lax.GatherScatterMode.PROMISE_IN_BOUNDS)


def _cbow_body(num_chunks, L, K, D,
               ctx_hbm, tgt_hbm, emb_hbm, tgtw_hbm, out_hbm,
               idxc_a, idxt_a, rows_ca, rows_ta, pooled_a, scores_a,
               sem_ca, sem_ta, sem_ia,
               idxc_b, idxt_b, rows_cb, rows_tb, pooled_b, scores_b,
               sem_cb, sem_tb, sem_ib,
               e0_buf, tscr):
  nd = D // LANES
  wid = lax.axis_index("s") * NC + lax.axis_index("c")
  base = wid * (num_chunks * CB)
  iota = lax.iota(jnp.int32, LANES)

  pltpu.sync_copy(emb_hbm.at[pl.ds(0, 1)], e0_buf)
  for r in range(OUTP):
    tscr[r] = jnp.zeros((LANES,), jnp.float32)

  def stage(chunk, idxc, idxt, half, sem_i):
    row0 = base + chunk * CB
    pltpu.async_copy(ctx_hbm.at[pl.ds(row0, CB)],
                     idxc.at[pl.ds(half * CB, CB)], sem_i)
    pltpu.async_copy(tgt_hbm.at[pl.ds(row0, CB)],
                     idxt.at[pl.ds(half * CB, CB)], sem_i)

  def drain_idx(idxc, idxt, half, sem_i):
    pltpu.make_async_copy(ctx_hbm.at[pl.ds(base, CB)],
                          idxc.at[pl.ds(half * CB, CB)], sem_i).wait()
    pltpu.make_async_copy(tgt_hbm.at[pl.ds(base, CB)],
                          idxt.at[pl.ds(half * CB, CB)], sem_i).wait()

  def fire(idxc, idxt, half, rows_c, rows_t, sem_c, sem_t):
    for j in range(CB):
      pltpu.async_copy(emb_hbm.at[idxc.at[half * CB + j]],
                       rows_c.at[pl.ds(j * L, L)], sem_c)
      pltpu.async_copy(tgtw_hbm.at[idxt.at[half * CB + j]],
                       rows_t.at[pl.ds(j * K, K)], sem_t)

  def drain(idxc, idxt, half, rows_c, rows_t, sem_c, sem_t):
    for j in range(CB):
      pltpu.make_async_copy(emb_hbm.at[idxc.at[half * CB + j]],
                            rows_c.at[pl.ds(j * L, L)], sem_c).wait()
      pltpu.make_async_copy(tgtw_hbm.at[idxt.at[half * CB + j]],
                            rows_t.at[pl.ds(j * K, K)], sem_t).wait()

  def compute(chunk, idxc, half, rows_c, rows_t, pooled, scores):
    def pool_b(b, carry2):
      rowvec = jnp.full((LANES,), half * CB + b, jnp.int32)
      bvec = jnp.full((LANES,), b, jnp.int32)
      zc = jnp.zeros((LANES,), jnp.float32)
      one = jnp.float32(1.0)
      zero = jnp.float32(0.0)
      for g in range(L // LANES):
        v = plsc.load_gather(idxc, [rowvec, iota + g * LANES])
        zc = zc + jnp.where(v == 0, one, zero)
      rem = L % LANES
      if rem:
        v = plsc.load_gather(idxc, [rowvec, iota + (L - LANES)])
        zc = zc + jnp.where((v == 0) & (iota >= LANES - rem), one, zero)
      nz = zc
      for sh in (8, 4, 2, 1):
        nz = nz + _lane_shuffle(nz, iota ^ sh)

      def l_body(l, acc):
        r0 = jnp.full((LANES,), b * L + 2 * l, jnp.int32)
        r1 = r0 + 1
        return tuple(
            acc[cc]
            + plsc.load_gather(rows_c, [r0, iota + cc * LANES])
            + plsc.load_gather(rows_c, [r1, iota + cc * LANES])
            for cc in range(nd))
      acc = lax.fori_loop(0, L // 2, l_body,
                          tuple(jnp.zeros((LANES,), jnp.float32)
                                for _ in range(nd)))
      scale = jnp.float32(1.0 / L)
      for cc in range(nd):
        e0 = e0_buf[0, pl.ds(cc * LANES, LANES)]
        plsc.store_scatter(pooled, [bvec, iota + cc * LANES],
                           (acc[cc] - nz * e0) * scale)
      return carry2

    lax.fori_loop(0, CB, pool_b, 0)

    def score_b(b, carry2):
      bvec = jnp.full((LANES,), b, jnp.int32)
      p = [plsc.load_gather(pooled, [bvec, iota + cc * LANES])
           for cc in range(nd)]
      for k in range(K):
        rvec = jnp.full((LANES,), b * K + k, jnp.int32)
        acc = plsc.load_gather(rows_t, [rvec, iota]) * p[0]
        for cc in range(1, nd):
          acc = acc + plsc.load_gather(
              rows_t, [rvec, iota + cc * LANES]) * p[cc]
        tscr[k] = acc
      for g in range(OUTP // LANES):
        ridx = iota + (g * LANES)
        s = plsc.load_gather(tscr, [ridx, jnp.zeros((LANES,), jnp.int32)])
        for cc in range(1, LANES):
          s = s + plsc.load_gather(
              tscr, [ridx, jnp.full((LANES,), cc, jnp.int32)])
        plsc.store_scatter(scores, [bvec, iota + g * LANES], s)
      return carry2

    lax.fori_loop(0, CB, score_b, 0)
    pltpu.sync_copy(scores, out_hbm.at[pl.ds(base + chunk * CB, CB)])

  zero = jnp.int32(0)
  stage(zero, idxc_a, idxt_a, zero, sem_ia)
  drain_idx(idxc_a, idxt_a, zero, sem_ia)
  fire(idxc_a, idxt_a, zero, rows_ca, rows_ta, sem_ca, sem_ta)
  stage(jnp.int32(1), idxc_b, idxt_b, zero, sem_ib)

  last = jnp.int32(num_chunks - 1)

  def pair(i, carry):
    ca = 2 * i
    cb = 2 * i + 1
    h = i & 1
    hn = 1 - h
    drain_idx(idxc_b, idxt_b, h, sem_ib)
    fire(idxc_b, idxt_b, h, rows_cb, rows_tb, sem_cb, sem_tb)
    stage(jnp.minimum(ca + 2, last), idxc_a, idxt_a, hn, sem_ia)
    drain(idxc_a, idxt_a, h, rows_ca, rows_ta, sem_ca, sem_ta)
    compute(ca, idxc_a, h, rows_ca, rows_ta, pooled_a, scores_a)
    drain_idx(idxc_a, idxt_a, hn, sem_ia)
    fire(idxc_a, idxt_a, hn, rows_ca, rows_ta, sem_ca, sem_ta)
    stage(jnp.minimum(cb + 2, last), idxc_b, idxt_b, hn, sem_ib)
    drain(idxc_b, idxt_b, h, rows_cb, rows_tb, sem_cb, sem_tb)
    compute(cb, idxc_b, h, rows_cb, rows_tb, pooled_b, scores_b)
    return carry

  lax.fori_loop(0, num_chunks // 2, pair, 0)
  hlast = jnp.int32((num_chunks // 2) & 1)
  drain(idxc_a, idxt_a, hlast, rows_ca, rows_ta, sem_ca, sem_ta)
  drain_idx(idxc_b, idxt_b, hlast, sem_ib)


def kernel(context, target, emb_weight, tgt_weight):
  B, L = context.shape
  _, K = target.shape
  V, D = emb_weight.shape
  assert B % (NW * CB * 2) == 0
  num_chunks = B // (NW * CB)

  body = functools.partial(_cbow_body, num_chunks, L, K, D)
  mesh = plsc.VectorSubcoreMesh(core_axis_name="c", subcore_axis_name="s")

  def set_scratch():
    return [
        pltpu.VMEM((2 * CB, L), jnp.int32),
        pltpu.VMEM((2 * CB, K), jnp.int32),
        pltpu.VMEM((CB * L, D), jnp.float32),
        pltpu.VMEM((CB * K, D), jnp.float32),
        pltpu.VMEM((CB, D), jnp.float32),
        pltpu.VMEM((CB, OUTP), jnp.float32),
        pltpu.SemaphoreType.DMA,
        pltpu.SemaphoreType.DMA,
        pltpu.SemaphoreType.DMA,
    ]

  run = pl.kernel(
      body,
      out_type=jax.ShapeDtypeStruct((B, OUTP), jnp.float32),
      mesh=mesh,
      compiler_params=pltpu.CompilerParams(needs_layout_passes=False),
      scratch_types=set_scratch() + set_scratch() + [
          pltpu.VMEM((1, D), jnp.float32),
          pltpu.VMEM((OUTP, LANES), jnp.float32),
      ],
  )
  out = run(context.astype(jnp.int32), target.astype(jnp.int32),
            emb_weight, tgt_weight)
  return out[:, :K]
